# baseline (speedup 1.0000x reference)
import functools

import jax
import jax.numpy as jnp
from jax.experimental import pallas as pl
from jax.experimental.pallas import tpu as pltpu
from jax.experimental.pallas import tpu_sc as plsc

N = 38333
F = 898
D = 128
E = 613328
NPAD = 38400
EPAD = 622592
ECH = EPAD // 128
DUMMY = NPAD - 1
RB = 2400
NBLK = NPAD // RB
ROWS_PER_TILE = NPAD // 16

_MESH = dict(core_axis_name="c", subcore_axis_name="s", num_cores=2,
             num_subcores=16)


def _leaky(v):
    return jnp.where(v >= 0, v, 0.01 * v)


def _make_sc_scatter(mode_a):
    W = 16
    NB = 4
    CPT = 304 if mode_a else 152
    mesh = plsc.VectorSubcoreMesh(**_MESH)

    def body(ga, gb, srcp, dstp, zrows, out, idx_s, idx_d, rows, acc,
             gsem, ssem):
        c = jax.lax.axis_index("c")
        s = jax.lax.axis_index("s")
        base = s * ROWS_PER_TILE
        pltpu.sync_copy(zrows, acc.at[pl.ds(base, ROWS_PER_TILE)])
        plsc.subcore_barrier()
        if mode_a:
            chunk0 = s * CPT
        else:
            chunk0 = (s * 2 + c) * CPT
        pltpu.sync_copy(srcp.at[pl.ds(chunk0, CPT)], idx_s)
        pltpu.sync_copy(dstp.at[pl.ds(chunk0, CPT)], idx_d)

        def issue_gather(j, b):
            @pl.when(c == 0)
            def _():
                pltpu.async_copy(ga.at[idx_s.at[j]], rows.at[b], gsem.at[b])

            @pl.when(c == 1)
            def _():
                pltpu.async_copy(gb.at[idx_s.at[j]], rows.at[b], gsem.at[b])

        def wait_gather(j, b):
            pltpu.make_async_copy(ga.at[idx_s.at[j]], rows.at[b],
                                  gsem.at[b]).wait()

        def wait_scatter(j, b):
            pltpu.make_async_copy(rows.at[b], acc.at[idx_d.at[j]],
                                  ssem.at[b]).wait()

        for p in range(NB - 1):
            issue_gather(p, p)

        def loop_body(j, carry):
            b = jax.lax.rem(j, NB)

            @pl.when(j + NB - 1 < CPT)
            def _():
                @pl.when(j >= 1)
                def _():
                    wait_scatter(j - 1, jax.lax.rem(j - 1, NB))
                issue_gather(j + NB - 1, jax.lax.rem(j + NB - 1, NB))

            wait_gather(j, b)
            pltpu.async_copy(rows.at[b], acc.at[idx_d.at[j]], ssem.at[b],
                             add=True)
            return carry

        jax.lax.fori_loop(0, CPT, loop_body, 0)
        for p in range(CPT - NB, CPT):
            wait_scatter(p, p % NB)
        plsc.subcore_barrier()

        @pl.when(c == 0)
        def _():
            pltpu.sync_copy(acc.at[pl.ds(base, ROWS_PER_TILE)],
                            out.at[0].at[pl.ds(base, ROWS_PER_TILE)])

        @pl.when(c == 1)
        def _():
            pltpu.sync_copy(acc.at[pl.ds(base, ROWS_PER_TILE)],
                            out.at[1].at[pl.ds(base, ROWS_PER_TILE)])

    return pl.kernel(
        body,
        out_type=jax.ShapeDtypeStruct((2, NPAD, W), jnp.float32),
        mesh=mesh,
        compiler_params=pltpu.CompilerParams(use_tc_tiling_on_sc=False),
        scratch_types=[
            pltpu.VMEM((CPT, 128), jnp.int32),
            pltpu.VMEM((CPT, 128), jnp.int32),
            pltpu.VMEM((4, 128, W), jnp.float32),
            pltpu.VMEM_SHARED((NPAD, W), jnp.float32),
            pltpu.SemaphoreType.DMA((4,)),
            pltpu.SemaphoreType.DMA((4,)),
        ],
    )


def _make_sc_scatter_l1():
    W = 16
    NB = 4
    CPT = 304
    mesh = plsc.VectorSubcoreMesh(**_MESH)

    def body(q0, q1, q2, q3, srcp, dstp, zrows, out, idx_s, idx_d, rows, acc,
             gsem, ssem):
        c = jax.lax.axis_index("c")
        s = jax.lax.axis_index("s")
        base = s * ROWS_PER_TILE
        chunk0 = s * CPT
        pltpu.sync_copy(srcp.at[pl.ds(chunk0, CPT)], idx_s)
        pltpu.sync_copy(dstp.at[pl.ds(chunk0, CPT)], idx_d)

        def wait_gather(j, b):
            pltpu.make_async_copy(q0.at[idx_s.at[j]], rows.at[b],
                                  gsem.at[b]).wait()

        def wait_scatter(j, b):
            pltpu.make_async_copy(rows.at[b], acc.at[idx_d.at[j]],
                                  ssem.at[b]).wait()

        for p in range(2):
            ga = (q0, q2)[p]
            gb = (q1, q3)[p]

            def issue_gather(j, b, ga=ga, gb=gb):
                @pl.when(c == 0)
                def _():
                    pltpu.async_copy(ga.at[idx_s.at[j]], rows.at[b],
                                     gsem.at[b])

                @pl.when(c == 1)
                def _():
                    pltpu.async_copy(gb.at[idx_s.at[j]], rows.at[b],
                                     gsem.at[b])

            plsc.subcore_barrier()
            pltpu.sync_copy(zrows, acc.at[pl.ds(base, ROWS_PER_TILE)])
            plsc.subcore_barrier()

            for q in range(NB - 1):
                issue_gather(q, q)

            def loop_body(j, carry):
                b = jax.lax.rem(j, NB)

                @pl.when(j + NB - 1 < CPT)
                def _():
                    @pl.when(j >= 1)
                    def _():
                        wait_scatter(j - 1, jax.lax.rem(j - 1, NB))
                    issue_gather(j + NB - 1, jax.lax.rem(j + NB - 1, NB))

                wait_gather(j, b)
                pltpu.async_copy(rows.at[b], acc.at[idx_d.at[j]],
                                 ssem.at[b], add=True)
                return carry

            jax.lax.fori_loop(0, CPT, loop_body, 0)
            for q in range(CPT - NB, CPT):
                wait_scatter(q, q % NB)
            plsc.subcore_barrier()

            @pl.when(c == 0)
            def _():
                pltpu.sync_copy(acc.at[pl.ds(base, ROWS_PER_TILE)],
                                out.at[2 * p].at[pl.ds(base, ROWS_PER_TILE)])

            @pl.when(c == 1)
            def _():
                pltpu.sync_copy(
                    acc.at[pl.ds(base, ROWS_PER_TILE)],
                    out.at[2 * p + 1].at[pl.ds(base, ROWS_PER_TILE)])

    return pl.kernel(
        body,
        out_type=jax.ShapeDtypeStruct((4, NPAD, W), jnp.float32),
        mesh=mesh,
        compiler_params=pltpu.CompilerParams(use_tc_tiling_on_sc=False),
        scratch_types=[
            pltpu.VMEM((CPT, 128), jnp.int32),
            pltpu.VMEM((CPT, 128), jnp.int32),
            pltpu.VMEM((4, 128, W), jnp.float32),
            pltpu.VMEM_SHARED((NPAD, W), jnp.float32),
            pltpu.SemaphoreType.DMA((4,)),
            pltpu.SemaphoreType.DMA((4,)),
        ],
    )


def _rowmask(i, rows, cols):
    rid = i * RB + jax.lax.broadcasted_iota(jnp.int32, (rows, cols), 0)
    return rid < N


def _tc1(x, W1, newcol3, degp):
    def body(x_ref, w_ref, nc_ref, dp_ref,
             o0_ref, o1_ref, o2_ref, o3_ref):
        i = pl.program_id(0)
        xb = x_ref[...]
        colid = jax.lax.broadcasted_iota(jnp.int32, (RB, F), 1)
        xb = jnp.where(colid == 0, nc_ref[0, 0, :][:, None], xb)
        h = jnp.dot(xb, w_ref[...], preferred_element_type=jnp.float32)
        dp = dp_ref[...]
        dinv = jax.lax.rsqrt(dp[0][:, :1] + dp[1][:, :1] + 1.0)
        g = h * dinv
        g = jnp.where(_rowmask(i, RB, 64), g, 0.0)
        o0_ref[...] = g[:, 0:16]
        o1_ref[...] = g[:, 16:32]
        o2_ref[...] = g[:, 32:48]
        o3_ref[...] = g[:, 48:64]

    return pl.pallas_call(
        body,
        grid=(NBLK,),
        in_specs=[
            pl.BlockSpec((RB, F), lambda i: (i, 0)),
            pl.BlockSpec((F, 64), lambda i: (0, 0)),
            pl.BlockSpec((1, 1, RB), lambda i: (i, 0, 0)),
            pl.BlockSpec((2, RB, 16), lambda i: (0, i, 0)),
        ],
        out_specs=[pl.BlockSpec((RB, 16), lambda i: (i, 0))] * 4,
        out_shape=[jax.ShapeDtypeStruct((NPAD, 16), jnp.float32)] * 4,
    )(x, W1, newcol3, degp)


def _tc2(S1, q0, q1, q2, q3, degp, b1r, W2):
    def body(s_ref, q0_ref, q1_ref, q2_ref, q3_ref,
             dp_ref, b_ref, w_ref, o0_ref, o1_ref):
        i = pl.program_id(0)
        dp = dp_ref[...]
        dinv = jax.lax.rsqrt(dp[0][:, :1] + dp[1][:, :1] + 1.0)
        agg = jnp.concatenate(
            [s_ref[0] + q0_ref[...], s_ref[1] + q1_ref[...],
             s_ref[2] + q2_ref[...], s_ref[3] + q3_ref[...]], axis=1)
        f1 = _leaky(dinv * agg + b_ref[0, :][None, :])
        h2 = jnp.dot(f1, w_ref[...], preferred_element_type=jnp.float32)
        g2 = jnp.where(_rowmask(i, RB, 32), h2 * dinv, 0.0)
        o0_ref[...] = g2[:, 0:16]
        o1_ref[...] = g2[:, 16:32]

    return pl.pallas_call(
        body,
        grid=(NBLK,),
        in_specs=[
            pl.BlockSpec((4, RB, 16), lambda i: (0, i, 0)),
            pl.BlockSpec((RB, 16), lambda i: (i, 0)),
            pl.BlockSpec((RB, 16), lambda i: (i, 0)),
            pl.BlockSpec((RB, 16), lambda i: (i, 0)),
            pl.BlockSpec((RB, 16), lambda i: (i, 0)),
            pl.BlockSpec((2, RB, 16), lambda i: (0, i, 0)),
            pl.BlockSpec((1, 64), lambda i: (0, 0)),
            pl.BlockSpec((64, 32), lambda i: (0, 0)),
        ],
        out_specs=[pl.BlockSpec((RB, 16), lambda i: (i, 0))] * 2,
        out_shape=[jax.ShapeDtypeStruct((NPAD, 16), jnp.float32)] * 2,
    )(S1, q0, q1, q2, q3, degp, b1r, W2)


def _tc_mid(S, h0, h1, degp, br, Wn, residual):
    wout = Wn.shape[1]

    def body(s_ref, h0_ref, h1_ref, dp_ref, b_ref, w_ref, *o_refs):
        i = pl.program_id(0)
        dp = dp_ref[...]
        dinv = jax.lax.rsqrt(dp[0][:, :1] + dp[1][:, :1] + 1.0)
        agg = jnp.concatenate(
            [s_ref[0] + h0_ref[...], s_ref[1] + h1_ref[...]], axis=1)
        t = dinv * agg + b_ref[0, :][None, :]
        f = _leaky(t) + t if residual else _leaky(t)
        h = jnp.dot(f, w_ref[...], preferred_element_type=jnp.float32)
        g2 = h * dinv
        if wout == 1:
            g2 = jnp.broadcast_to(g2[:, :1], (RB, 16))
            o_refs[0][...] = jnp.where(_rowmask(i, RB, 16), g2, 0.0)
        else:
            g2 = jnp.where(_rowmask(i, RB, 32), g2, 0.0)
            o_refs[0][...] = g2[:, 0:16]
            o_refs[1][...] = g2[:, 16:32]

    n_out = 1 if wout == 1 else 2
    return pl.pallas_call(
        body,
        grid=(NBLK,),
        in_specs=[
            pl.BlockSpec((2, RB, 16), lambda i: (0, i, 0)),
            pl.BlockSpec((RB, 16), lambda i: (i, 0)),
            pl.BlockSpec((RB, 16), lambda i: (i, 0)),
            pl.BlockSpec((2, RB, 16), lambda i: (0, i, 0)),
            pl.BlockSpec((1, 32), lambda i: (0, 0)),
            pl.BlockSpec((32, wout), lambda i: (0, 0)),
        ],
        out_specs=[pl.BlockSpec((RB, 16), lambda i: (i, 0))] * n_out,
        out_shape=[jax.ShapeDtypeStruct((NPAD, 16), jnp.float32)] * n_out,
    )(S, h0, h1, degp, br, Wn)


def _tc_final(S5, g5, degp, b5r, fc1_W, fc1_br, fc2_W, fc2_br):
    def body(s_ref, g_ref, dp_ref, b5_ref, w1_ref, b1_ref,
             w2_ref, b2_ref, o_ref, acc_ref):
        i = pl.program_id(0)
        dp = dp_ref[...]
        dinv = jax.lax.rsqrt(dp[0][:, :1] + dp[1][:, :1] + 1.0)
        t = dinv * (s_ref[0] + s_ref[1] + g_ref[...]) + b5_ref[0, 0]
        f5 = _leaky(t)[:, :1]
        m = _rowmask(i, RB, 1)
        f5 = jnp.where(m, f5, 0.0)
        w1b = jnp.where(_rowmask(i, RB, 128), w1_ref[...], 0.0)
        part = jax.lax.dot_general(
            f5, w1b, (((0,), (0,)), ((), ())),
            preferred_element_type=jnp.float32)

        @pl.when(i == 0)
        def _():
            acc_ref[...] = jnp.zeros_like(acc_ref)

        acc_ref[...] += part

        @pl.when(i == NBLK - 1)
        def _():
            h = jnp.maximum(acc_ref[...] + b1_ref[...], 0.0)
            o = jnp.dot(h, w2_ref[...], preferred_element_type=jnp.float32)
            o_ref[...] = jnp.maximum(o + b2_ref[...], 0.0)

    return pl.pallas_call(
        body,
        grid=(NBLK,),
        in_specs=[
            pl.BlockSpec((2, RB, 16), lambda i: (0, i, 0)),
            pl.BlockSpec((RB, 16), lambda i: (i, 0)),
            pl.BlockSpec((2, RB, 16), lambda i: (0, i, 0)),
            pl.BlockSpec((1, 1), lambda i: (0, 0)),
            pl.BlockSpec((RB, 128), lambda i: (i, 0)),
            pl.BlockSpec((1, 128), lambda i: (0, 0)),
            pl.BlockSpec((128, 128), lambda i: (0, 0)),
            pl.BlockSpec((1, 128), lambda i: (0, 0)),
        ],
        out_specs=pl.BlockSpec((1, 128), lambda i: (0, 0)),
        out_shape=jax.ShapeDtypeStruct((1, 128), jnp.float32),
        scratch_shapes=[pltpu.VMEM((1, 128), jnp.float32)],
    )(S5, g5, degp, b5r, fc1_W, fc1_br, fc2_W, fc2_br)


def _sc_deg_emb():
    CPT = 152
    mesh = plsc.VectorSubcoreMesh(**_MESH)

    def body(dstp, poi_idx, emb, zrows, ones_h, degp, emb_out,
             idx_d, ones_v, pidx_v, embbuf, acc, gsem, ssem):
        c = jax.lax.axis_index("c")
        s = jax.lax.axis_index("s")
        base = s * ROWS_PER_TILE
        pltpu.sync_copy(zrows, acc.at[pl.ds(base, ROWS_PER_TILE)])
        plsc.subcore_barrier()
        chunk0 = (s * 2 + c) * CPT
        pltpu.sync_copy(dstp.at[pl.ds(chunk0, CPT)], idx_d)
        pltpu.sync_copy(ones_h, ones_v)

        def start_body(j, carry):
            pltpu.async_copy(ones_v, acc.at[idx_d.at[j]], ssem, add=True)
            return carry

        jax.lax.fori_loop(0, CPT, start_body, 0)

        def wait_body(j, carry):
            pltpu.make_async_copy(ones_v, acc.at[idx_d.at[j]], ssem).wait()
            return carry

        jax.lax.fori_loop(0, CPT, wait_body, 0)

        @pl.when((c == 0) & (s == 0))
        def _():
            pltpu.sync_copy(poi_idx, pidx_v)

            def emb_body(k, carry):
                pltpu.async_copy(emb.at[pidx_v.at[k]], embbuf, gsem).wait()
                pltpu.sync_copy(embbuf, emb_out.at[pl.ds(k * 128, 128)])
                return carry

            jax.lax.fori_loop(0, 3, emb_body, 0)

        plsc.subcore_barrier()

        @pl.when(c == 0)
        def _():
            pltpu.sync_copy(acc.at[pl.ds(base, ROWS_PER_TILE)],
                            degp.at[0].at[pl.ds(base, ROWS_PER_TILE)])

        @pl.when(c == 1)
        def _():
            pltpu.sync_copy(acc.at[pl.ds(base, ROWS_PER_TILE)],
                            degp.at[1].at[pl.ds(base, ROWS_PER_TILE)])

    return pl.kernel(
        body,
        out_type=[
            jax.ShapeDtypeStruct((2, NPAD, 16), jnp.float32),
            jax.ShapeDtypeStruct((384, 128), jnp.float32),
        ],
        mesh=mesh,
        compiler_params=pltpu.CompilerParams(use_tc_tiling_on_sc=False),
        scratch_types=[
            pltpu.VMEM((CPT, 128), jnp.int32),
            pltpu.VMEM((128, 16), jnp.float32),
            pltpu.VMEM((3, 128), jnp.int32),
            pltpu.VMEM((128, 128), jnp.float32),
            pltpu.VMEM_SHARED((NPAD, 16), jnp.float32),
            pltpu.SemaphoreType.DMA,
            pltpu.SemaphoreType.DMA,
        ],
    )


def kernel(x, edge_index, mask, emb_table, W1, b1, W2, b2, W3, b3, W4, b4,
           W5, b5, fc1_W, fc1_b, fc2_W, fc2_b):
    del mask

    src = edge_index[0]
    dst = edge_index[1]
    srcp = jnp.concatenate(
        [src, jnp.zeros((EPAD - E,), jnp.int32)]).reshape(ECH, 128)
    dstp = jnp.concatenate(
        [dst, jnp.full((EPAD - E,), DUMMY, jnp.int32)]).reshape(ECH, 128)
    npoi = (N + D - 1) // D
    poi = x[:npoi, 0].astype(jnp.int32)
    poi3 = jnp.concatenate(
        [poi, jnp.zeros((384 - npoi,), jnp.int32)]).reshape(3, 128)
    zeros16 = jnp.zeros((ROWS_PER_TILE, 16), jnp.float32)
    ones16 = jnp.ones((128, 16), jnp.float32)

    degp, emb_rows = _sc_deg_emb()(dstp, poi3, emb_table, zeros16, ones16)
    newcol = emb_rows.reshape(-1)[:N]
    newcol3 = jnp.concatenate(
        [newcol, jnp.zeros((NPAD - N,), jnp.float32)]).reshape(NBLK, 1, RB)

    q0, q1, q2, q3 = _tc1(x, W1, newcol3, degp)
    scat_a = _make_sc_scatter(True)
    S1 = _make_sc_scatter_l1()(q0, q1, q2, q3, srcp, dstp, zeros16)
    g2h = _tc2(S1, q0, q1, q2, q3, degp, b1.reshape(1, 64), W2)

    S2 = scat_a(g2h[0], g2h[1], srcp, dstp, zeros16)
    g3h = _tc_mid(S2, g2h[0], g2h[1], degp, b2.reshape(1, 32), W3,
                  residual=False)
    S3 = scat_a(g3h[0], g3h[1], srcp, dstp, zeros16)
    g4h = _tc_mid(S3, g3h[0], g3h[1], degp, b3.reshape(1, 32), W4,
                  residual=True)
    S4 = scat_a(g4h[0], g4h[1], srcp, dstp, zeros16)
    (g5,) = _tc_mid(S4, g4h[0], g4h[1], degp, b4.reshape(1, 32), W5,
                    residual=True)

    S5 = _make_sc_scatter(False)(g5, g5, srcp, dstp, zeros16)
    out = _tc_final(S5, g5, degp, b5.reshape(1, 1), fc1_W,
                    fc1_b.reshape(1, 128), fc2_W, fc2_b.reshape(1, 128))
    return out.reshape(128)


# speedup vs baseline: 1.0112x; 1.0112x over previous
import functools

import jax
import jax.numpy as jnp
from jax.experimental import pallas as pl
from jax.experimental.pallas import tpu as pltpu
from jax.experimental.pallas import tpu_sc as plsc

N = 38333
F = 898
D = 128
E = 613328
NPAD = 38400
EPAD = 622592
ECH = EPAD // 128
DUMMY = NPAD - 1
RB = 2400
NBLK = NPAD // RB
ROWS_PER_TILE = NPAD // 16

_MESH = dict(core_axis_name="c", subcore_axis_name="s", num_cores=2,
             num_subcores=16)


def _leaky(v):
    return jnp.where(v >= 0, v, 0.01 * v)


def _make_sc_scatter(mode_a):
    W = 16
    NB = 4
    CPT = 304 if mode_a else 152
    mesh = plsc.VectorSubcoreMesh(**_MESH)

    def body(ga, gb, srcp, dstp, zrows, out, idx_s, idx_d, rows, acc,
             gsem, ssem):
        c = jax.lax.axis_index("c")
        s = jax.lax.axis_index("s")
        base = s * ROWS_PER_TILE
        pltpu.sync_copy(zrows, acc.at[pl.ds(base, ROWS_PER_TILE)])
        plsc.subcore_barrier()
        if mode_a:
            chunk0 = s * CPT
        else:
            chunk0 = (s * 2 + c) * CPT
        pltpu.sync_copy(srcp.at[pl.ds(chunk0, CPT)], idx_s)
        pltpu.sync_copy(dstp.at[pl.ds(chunk0, CPT)], idx_d)

        def issue_gather(j, b):
            @pl.when(c == 0)
            def _():
                pltpu.async_copy(ga.at[idx_s.at[j]], rows.at[b], gsem.at[b])

            @pl.when(c == 1)
            def _():
                pltpu.async_copy(gb.at[idx_s.at[j]], rows.at[b], gsem.at[b])

        def wait_gather(j, b):
            pltpu.make_async_copy(ga.at[idx_s.at[j]], rows.at[b],
                                  gsem.at[b]).wait()

        def wait_scatter(j, b):
            pltpu.make_async_copy(rows.at[b], acc.at[idx_d.at[j]],
                                  ssem.at[b]).wait()

        for p in range(NB - 1):
            issue_gather(p, p)

        def loop_body(j, carry):
            b = jax.lax.rem(j, NB)

            @pl.when(j + NB - 1 < CPT)
            def _():
                @pl.when(j >= 1)
                def _():
                    wait_scatter(j - 1, jax.lax.rem(j - 1, NB))
                issue_gather(j + NB - 1, jax.lax.rem(j + NB - 1, NB))

            wait_gather(j, b)
            pltpu.async_copy(rows.at[b], acc.at[idx_d.at[j]], ssem.at[b],
                             add=True)
            return carry

        jax.lax.fori_loop(0, CPT, loop_body, 0)
        for p in range(CPT - NB, CPT):
            wait_scatter(p, p % NB)
        plsc.subcore_barrier()

        @pl.when(c == 0)
        def _():
            pltpu.sync_copy(acc.at[pl.ds(base, ROWS_PER_TILE)],
                            out.at[0].at[pl.ds(base, ROWS_PER_TILE)])

        @pl.when(c == 1)
        def _():
            pltpu.sync_copy(acc.at[pl.ds(base, ROWS_PER_TILE)],
                            out.at[1].at[pl.ds(base, ROWS_PER_TILE)])

    return pl.kernel(
        body,
        out_type=jax.ShapeDtypeStruct((2, NPAD, W), jnp.float32),
        mesh=mesh,
        compiler_params=pltpu.CompilerParams(use_tc_tiling_on_sc=False),
        scratch_types=[
            pltpu.VMEM((CPT, 128), jnp.int32),
            pltpu.VMEM((CPT, 128), jnp.int32),
            pltpu.VMEM((4, 128, W), jnp.float32),
            pltpu.VMEM_SHARED((NPAD, W), jnp.float32),
            pltpu.SemaphoreType.DMA((4,)),
            pltpu.SemaphoreType.DMA((4,)),
        ],
    )


def _rowmask(i, rows, cols):
    rid = i * RB + jax.lax.broadcasted_iota(jnp.int32, (rows, cols), 0)
    return rid < N


def _tc1(x, W1, newcol3, degp):
    def body(x_ref, w_ref, nc_ref, dp_ref,
             o0_ref, o1_ref, o2_ref, o3_ref):
        i = pl.program_id(0)
        xb = x_ref[...]
        colid = jax.lax.broadcasted_iota(jnp.int32, (RB, F), 1)
        xb = jnp.where(colid == 0, nc_ref[0, 0, :][:, None], xb)
        h = jnp.dot(xb, w_ref[...], preferred_element_type=jnp.float32)
        dp = dp_ref[...]
        dinv = jax.lax.rsqrt(dp[0][:, :1] + dp[1][:, :1] + 1.0)
        g = h * dinv
        g = jnp.where(_rowmask(i, RB, 64), g, 0.0)
        o0_ref[...] = g[:, 0:16]
        o1_ref[...] = g[:, 16:32]
        o2_ref[...] = g[:, 32:48]
        o3_ref[...] = g[:, 48:64]

    return pl.pallas_call(
        body,
        grid=(NBLK,),
        in_specs=[
            pl.BlockSpec((RB, F), lambda i: (i, 0)),
            pl.BlockSpec((F, 64), lambda i: (0, 0)),
            pl.BlockSpec((1, 1, RB), lambda i: (i, 0, 0)),
            pl.BlockSpec((2, RB, 16), lambda i: (0, i, 0)),
        ],
        out_specs=[pl.BlockSpec((RB, 16), lambda i: (i, 0))] * 4,
        out_shape=[jax.ShapeDtypeStruct((NPAD, 16), jnp.float32)] * 4,
    )(x, W1, newcol3, degp)


def _tc2(S1a, S1b, q0, q1, q2, q3, degp, b1r, W2):
    def body(sa_ref, sb_ref, q0_ref, q1_ref, q2_ref, q3_ref,
             dp_ref, b_ref, w_ref, o0_ref, o1_ref):
        i = pl.program_id(0)
        dp = dp_ref[...]
        dinv = jax.lax.rsqrt(dp[0][:, :1] + dp[1][:, :1] + 1.0)
        agg = jnp.concatenate(
            [sa_ref[0] + q0_ref[...], sa_ref[1] + q1_ref[...],
             sb_ref[0] + q2_ref[...], sb_ref[1] + q3_ref[...]], axis=1)
        f1 = _leaky(dinv * agg + b_ref[0, :][None, :])
        h2 = jnp.dot(f1, w_ref[...], preferred_element_type=jnp.float32)
        g2 = jnp.where(_rowmask(i, RB, 32), h2 * dinv, 0.0)
        o0_ref[...] = g2[:, 0:16]
        o1_ref[...] = g2[:, 16:32]

    return pl.pallas_call(
        body,
        grid=(NBLK,),
        in_specs=[
            pl.BlockSpec((2, RB, 16), lambda i: (0, i, 0)),
            pl.BlockSpec((2, RB, 16), lambda i: (0, i, 0)),
            pl.BlockSpec((RB, 16), lambda i: (i, 0)),
            pl.BlockSpec((RB, 16), lambda i: (i, 0)),
            pl.BlockSpec((RB, 16), lambda i: (i, 0)),
            pl.BlockSpec((RB, 16), lambda i: (i, 0)),
            pl.BlockSpec((2, RB, 16), lambda i: (0, i, 0)),
            pl.BlockSpec((1, 64), lambda i: (0, 0)),
            pl.BlockSpec((64, 32), lambda i: (0, 0)),
        ],
        out_specs=[pl.BlockSpec((RB, 16), lambda i: (i, 0))] * 2,
        out_shape=[jax.ShapeDtypeStruct((NPAD, 16), jnp.float32)] * 2,
    )(S1a, S1b, q0, q1, q2, q3, degp, b1r, W2)


def _tc_mid(S, h0, h1, degp, br, Wn, residual):
    wout = Wn.shape[1]

    def body(s_ref, h0_ref, h1_ref, dp_ref, b_ref, w_ref, *o_refs):
        i = pl.program_id(0)
        dp = dp_ref[...]
        dinv = jax.lax.rsqrt(dp[0][:, :1] + dp[1][:, :1] + 1.0)
        agg = jnp.concatenate(
            [s_ref[0] + h0_ref[...], s_ref[1] + h1_ref[...]], axis=1)
        t = dinv * agg + b_ref[0, :][None, :]
        f = _leaky(t) + t if residual else _leaky(t)
        h = jnp.dot(f, w_ref[...], preferred_element_type=jnp.float32)
        g2 = h * dinv
        if wout == 1:
            g2 = jnp.broadcast_to(g2[:, :1], (RB, 16))
            o_refs[0][...] = jnp.where(_rowmask(i, RB, 16), g2, 0.0)
        else:
            g2 = jnp.where(_rowmask(i, RB, 32), g2, 0.0)
            o_refs[0][...] = g2[:, 0:16]
            o_refs[1][...] = g2[:, 16:32]

    n_out = 1 if wout == 1 else 2
    return pl.pallas_call(
        body,
        grid=(NBLK,),
        in_specs=[
            pl.BlockSpec((2, RB, 16), lambda i: (0, i, 0)),
            pl.BlockSpec((RB, 16), lambda i: (i, 0)),
            pl.BlockSpec((RB, 16), lambda i: (i, 0)),
            pl.BlockSpec((2, RB, 16), lambda i: (0, i, 0)),
            pl.BlockSpec((1, 32), lambda i: (0, 0)),
            pl.BlockSpec((32, wout), lambda i: (0, 0)),
        ],
        out_specs=[pl.BlockSpec((RB, 16), lambda i: (i, 0))] * n_out,
        out_shape=[jax.ShapeDtypeStruct((NPAD, 16), jnp.float32)] * n_out,
    )(S, h0, h1, degp, br, Wn)


def _tc_final(S5, g5, degp, b5r, fc1_W, fc1_br, fc2_W, fc2_br):
    def body(s_ref, g_ref, dp_ref, b5_ref, w1_ref, b1_ref,
             w2_ref, b2_ref, o_ref, acc_ref):
        i = pl.program_id(0)
        dp = dp_ref[...]
        dinv = jax.lax.rsqrt(dp[0][:, :1] + dp[1][:, :1] + 1.0)
        t = dinv * (s_ref[0] + s_ref[1] + g_ref[...]) + b5_ref[0, 0]
        f5 = _leaky(t)[:, :1]
        m = _rowmask(i, RB, 1)
        f5 = jnp.where(m, f5, 0.0)
        w1b = jnp.where(_rowmask(i, RB, 128), w1_ref[...], 0.0)
        part = jax.lax.dot_general(
            f5, w1b, (((0,), (0,)), ((), ())),
            preferred_element_type=jnp.float32)

        @pl.when(i == 0)
        def _():
            acc_ref[...] = jnp.zeros_like(acc_ref)

        acc_ref[...] += part

        @pl.when(i == NBLK - 1)
        def _():
            h = jnp.maximum(acc_ref[...] + b1_ref[...], 0.0)
            o = jnp.dot(h, w2_ref[...], preferred_element_type=jnp.float32)
            o_ref[...] = jnp.maximum(o + b2_ref[...], 0.0)

    return pl.pallas_call(
        body,
        grid=(NBLK,),
        in_specs=[
            pl.BlockSpec((2, RB, 16), lambda i: (0, i, 0)),
            pl.BlockSpec((RB, 16), lambda i: (i, 0)),
            pl.BlockSpec((2, RB, 16), lambda i: (0, i, 0)),
            pl.BlockSpec((1, 1), lambda i: (0, 0)),
            pl.BlockSpec((RB, 128), lambda i: (i, 0)),
            pl.BlockSpec((1, 128), lambda i: (0, 0)),
            pl.BlockSpec((128, 128), lambda i: (0, 0)),
            pl.BlockSpec((1, 128), lambda i: (0, 0)),
        ],
        out_specs=pl.BlockSpec((1, 128), lambda i: (0, 0)),
        out_shape=jax.ShapeDtypeStruct((1, 128), jnp.float32),
        scratch_shapes=[pltpu.VMEM((1, 128), jnp.float32)],
    )(S5, g5, degp, b5r, fc1_W, fc1_br, fc2_W, fc2_br)


def _sc_deg_emb():
    CPT = 152
    mesh = plsc.VectorSubcoreMesh(**_MESH)

    def body(dstp, poi_idx, emb, zrows, ones_h, degp, emb_out,
             idx_d, ones_v, pidx_v, embbuf, acc, gsem, ssem):
        c = jax.lax.axis_index("c")
        s = jax.lax.axis_index("s")
        base = s * ROWS_PER_TILE
        pltpu.sync_copy(zrows, acc.at[pl.ds(base, ROWS_PER_TILE)])
        plsc.subcore_barrier()
        chunk0 = (s * 2 + c) * CPT
        pltpu.sync_copy(dstp.at[pl.ds(chunk0, CPT)], idx_d)
        pltpu.sync_copy(ones_h, ones_v)

        def start_body(j, carry):
            pltpu.async_copy(ones_v, acc.at[idx_d.at[j]], ssem, add=True)
            return carry

        jax.lax.fori_loop(0, CPT, start_body, 0)

        def wait_body(j, carry):
            pltpu.make_async_copy(ones_v, acc.at[idx_d.at[j]], ssem).wait()
            return carry

        jax.lax.fori_loop(0, CPT, wait_body, 0)

        @pl.when((c == 0) & (s == 0))
        def _():
            pltpu.sync_copy(poi_idx, pidx_v)

            def emb_body(k, carry):
                pltpu.async_copy(emb.at[pidx_v.at[k]], embbuf, gsem).wait()
                pltpu.sync_copy(embbuf, emb_out.at[pl.ds(k * 128, 128)])
                return carry

            jax.lax.fori_loop(0, 3, emb_body, 0)

        plsc.subcore_barrier()

        @pl.when(c == 0)
        def _():
            pltpu.sync_copy(acc.at[pl.ds(base, ROWS_PER_TILE)],
                            degp.at[0].at[pl.ds(base, ROWS_PER_TILE)])

        @pl.when(c == 1)
        def _():
            pltpu.sync_copy(acc.at[pl.ds(base, ROWS_PER_TILE)],
                            degp.at[1].at[pl.ds(base, ROWS_PER_TILE)])

    return pl.kernel(
        body,
        out_type=[
            jax.ShapeDtypeStruct((2, NPAD, 16), jnp.float32),
            jax.ShapeDtypeStruct((384, 128), jnp.float32),
        ],
        mesh=mesh,
        compiler_params=pltpu.CompilerParams(use_tc_tiling_on_sc=False),
        scratch_types=[
            pltpu.VMEM((CPT, 128), jnp.int32),
            pltpu.VMEM((128, 16), jnp.float32),
            pltpu.VMEM((3, 128), jnp.int32),
            pltpu.VMEM((128, 128), jnp.float32),
            pltpu.VMEM_SHARED((NPAD, 16), jnp.float32),
            pltpu.SemaphoreType.DMA,
            pltpu.SemaphoreType.DMA,
        ],
    )


def kernel(x, edge_index, mask, emb_table, W1, b1, W2, b2, W3, b3, W4, b4,
           W5, b5, fc1_W, fc1_b, fc2_W, fc2_b):
    del mask

    src = edge_index[0]
    dst = edge_index[1]
    srcp = jnp.concatenate(
        [src, jnp.zeros((EPAD - E,), jnp.int32)]).reshape(ECH, 128)
    dstp = jnp.concatenate(
        [dst, jnp.full((EPAD - E,), DUMMY, jnp.int32)]).reshape(ECH, 128)
    npoi = (N + D - 1) // D
    poi = x[:npoi, 0].astype(jnp.int32)
    poi3 = jnp.concatenate(
        [poi, jnp.zeros((384 - npoi,), jnp.int32)]).reshape(3, 128)
    zeros16 = jnp.zeros((ROWS_PER_TILE, 16), jnp.float32)
    ones16 = jnp.ones((128, 16), jnp.float32)

    degp, emb_rows = _sc_deg_emb()(dstp, poi3, emb_table, zeros16, ones16)
    newcol = emb_rows.reshape(-1)[:N]
    newcol3 = jnp.concatenate(
        [newcol, jnp.zeros((NPAD - N,), jnp.float32)]).reshape(NBLK, 1, RB)

    q0, q1, q2, q3 = _tc1(x, W1, newcol3, degp)
    scat_a = _make_sc_scatter(True)
    S1a = scat_a(q0, q1, srcp, dstp, zeros16)
    S1b = scat_a(q2, q3, srcp, dstp, zeros16)
    g2h = _tc2(S1a, S1b, q0, q1, q2, q3, degp, b1.reshape(1, 64), W2)

    S2 = scat_a(g2h[0], g2h[1], srcp, dstp, zeros16)
    g3h = _tc_mid(S2, g2h[0], g2h[1], degp, b2.reshape(1, 32), W3,
                  residual=False)
    S3 = scat_a(g3h[0], g3h[1], srcp, dstp, zeros16)
    g4h = _tc_mid(S3, g3h[0], g3h[1], degp, b3.reshape(1, 32), W4,
                  residual=True)
    S4 = scat_a(g4h[0], g4h[1], srcp, dstp, zeros16)
    (g5,) = _tc_mid(S4, g4h[0], g4h[1], degp, b4.reshape(1, 32), W5,
                    residual=True)

    S5 = _make_sc_scatter(False)(g5, g5, srcp, dstp, zeros16)
    out = _tc_final(S5, g5, degp, b5.reshape(1, 1), fc1_W,
                    fc1_b.reshape(1, 128), fc2_W, fc2_b.reshape(1, 128))
    return out.reshape(128)
